# baseline (speedup 1.0000x reference)
import functools

import jax
import jax.numpy as jnp
from jax.experimental import pallas as pl
from jax.experimental.pallas import tpu as pltpu

_KB = 1000


def _block_best(q, k, base, total_k):
    scores = jax.lax.dot_general(
        q, k, (((1,), (1,)), ((), ())),
        preferred_element_type=jnp.float32,
        precision=jax.lax.Precision.DEFAULT,
    )
    k_norm = jnp.sqrt(jnp.sum(k * k, axis=1))
    sim = scores / k_norm[None, :]
    local_max = jnp.max(sim, axis=1, keepdims=True)
    lanes = jax.lax.broadcasted_iota(jnp.int32, sim.shape, 1)
    local_idx = jnp.min(
        jnp.where(sim == local_max, lanes, jnp.int32(total_k)),
        axis=1, keepdims=True,
    ) + base
    return local_max, local_idx


def _body(q_ref, ka_ref, kb_ref, idx_ref, score_ref, *, kb, nblk, total_k):
    j = pl.program_id(0)
    q = q_ref[...]

    max_a, idx_a = _block_best(q, ka_ref[...], (2 * j) * kb, total_k)
    max_b, idx_b = _block_best(q, kb_ref[...], (2 * j + 1) * kb, total_k)

    a_wins = max_a >= max_b
    local_max = jnp.where(a_wins, max_a, max_b)
    local_idx = jnp.where(a_wins, idx_a, idx_b)

    @pl.when(j == 0)
    def _init():
        score_ref[...] = local_max
        idx_ref[...] = local_idx

    @pl.when(j > 0)
    def _merge():
        prev = score_ref[...]
        better = local_max > prev
        score_ref[...] = jnp.where(better, local_max, prev)
        idx_ref[...] = jnp.where(better, local_idx, idx_ref[...])

    @pl.when(j == nblk - 1)
    def _finalize():
        q_norm = jnp.sqrt(jnp.sum(q * q, axis=1, keepdims=True))
        score_ref[...] = score_ref[...] / q_norm


@jax.jit
def kernel(queries, keys):
    q, d = queries.shape
    k, _ = keys.shape
    nblk = k // (2 * _KB)
    assert nblk * 2 * _KB == k

    body = functools.partial(_body, kb=_KB, nblk=nblk, total_k=k)
    idx2, score2 = pl.pallas_call(
        body,
        grid=(nblk,),
        in_specs=[
            pl.BlockSpec((q, d), lambda j: (0, 0)),
            pl.BlockSpec((_KB, d), lambda j: (2 * j, 0)),
            pl.BlockSpec((_KB, d), lambda j: (2 * j + 1, 0)),
        ],
        out_specs=[
            pl.BlockSpec((q, 1), lambda j: (0, 0)),
            pl.BlockSpec((q, 1), lambda j: (0, 0)),
        ],
        out_shape=[
            jax.ShapeDtypeStruct((q, 1), jnp.int32),
            jax.ShapeDtypeStruct((q, 1), jnp.float32),
        ],
        compiler_params=pltpu.CompilerParams(
            dimension_semantics=("arbitrary",),
        ),
    )(queries, keys, keys)
    return idx2.reshape(q), score2.reshape(q)


# speedup vs baseline: 1.0285x; 1.0285x over previous
import functools

import jax
import jax.numpy as jnp
from jax.experimental import pallas as pl
from jax.experimental.pallas import tpu as pltpu

_KB = 2000
_NBUF = 3


def _body(q_ref, k_hbm, idx_ref, score_ref, kbuf, sem, *, kb, nblk, total_k):
    q = q_ref[...]

    def copy(i, slot):
        return pltpu.make_async_copy(
            k_hbm.at[pl.ds(i * kb, kb), :], kbuf.at[slot], sem.at[slot])

    for i in range(_NBUF - 1):
        copy(i, i).start()

    neg_inf = jnp.full((q.shape[0], 1), -jnp.inf, dtype=jnp.float32)
    zero_i = jnp.zeros((q.shape[0], 1), dtype=jnp.int32)

    def step(i, carry):
        best_s, best_i = carry
        slot = jax.lax.rem(i, _NBUF)

        nxt = i + _NBUF - 1
        @pl.when(nxt < nblk)
        def _prefetch():
            copy(nxt, jax.lax.rem(nxt, _NBUF)).start()

        copy(i, slot).wait()
        k = kbuf[slot]

        scores = jax.lax.dot_general(
            q, k, (((1,), (1,)), ((), ())),
            preferred_element_type=jnp.float32,
            precision=jax.lax.Precision.DEFAULT,
        )
        k_norm = jnp.sqrt(jnp.sum(k * k, axis=1))
        sim = scores / k_norm[None, :]

        local_max = jnp.max(sim, axis=1, keepdims=True)
        lanes = jax.lax.broadcasted_iota(jnp.int32, sim.shape, 1)
        local_idx = jnp.min(
            jnp.where(sim == local_max, lanes, jnp.int32(total_k)),
            axis=1, keepdims=True,
        ) + i * kb

        better = local_max > best_s
        return (jnp.where(better, local_max, best_s),
                jnp.where(better, local_idx, best_i))

    best_s, best_i = jax.lax.fori_loop(0, nblk, step, (neg_inf, zero_i))

    q_norm = jnp.sqrt(jnp.sum(q * q, axis=1, keepdims=True))
    idx_ref[...] = best_i
    score_ref[...] = best_s / q_norm


@jax.jit
def kernel(queries, keys):
    q, d = queries.shape
    k, _ = keys.shape
    nblk = k // _KB
    assert nblk * _KB == k

    body = functools.partial(_body, kb=_KB, nblk=nblk, total_k=k)
    idx2, score2 = pl.pallas_call(
        body,
        in_specs=[
            pl.BlockSpec((q, d), lambda: (0, 0)),
            pl.BlockSpec(memory_space=pltpu.MemorySpace.HBM),
        ],
        out_specs=[
            pl.BlockSpec((q, 1), lambda: (0, 0)),
            pl.BlockSpec((q, 1), lambda: (0, 0)),
        ],
        out_shape=[
            jax.ShapeDtypeStruct((q, 1), jnp.int32),
            jax.ShapeDtypeStruct((q, 1), jnp.float32),
        ],
        scratch_shapes=[
            pltpu.VMEM((_NBUF, _KB, d), jnp.float32),
            pltpu.SemaphoreType.DMA((_NBUF,)),
        ],
    )(queries, keys)
    return idx2.reshape(q), score2.reshape(q)
